# baseline (speedup 1.0000x reference)
import functools

import jax
import jax.numpy as jnp
from jax import lax
from jax.experimental import pallas as pl
from jax.experimental.pallas import tpu as pltpu
from jax.experimental.pallas import tpu_sc as plsc

_EPS = 1e-10
_KS = 5
_COMMIT = 0.25

_NC = 2
_NS = 16
_NW = _NC * _NS
_LANES = 16
_CH = 128


def _tc_select(xt_ref, d_ref, sup_ref):
    C, TB = xt_ref.shape
    K = d_ref.shape[1]
    xbt = xt_ref[...]
    dmat = d_ref[...]
    corr = lax.dot_general(
        dmat, xbt, (((0,), (0,)), ((), ())),
        preferred_element_type=jnp.float32)
    a = jnp.abs(corr)
    kiota = lax.broadcasted_iota(jnp.int32, (1, K), 1)
    digits = jnp.concatenate([kiota // 32, kiota % 32], axis=0)
    digits_bf = digits.astype(jnp.bfloat16)
    idxs = []
    for _ in range(_KS):
        m = jnp.max(a, axis=0, keepdims=True)
        hit = a >= m
        hitf = hit.astype(jnp.bfloat16)
        dig = lax.dot_general(
            digits_bf, hitf, (((1,), (0,)), ((), ())),
            preferred_element_type=jnp.float32)
        idx = dig[0:1, :].astype(jnp.int32) * 32 + dig[1:2, :].astype(jnp.int32)
        idxs.append(idx)
        a = jnp.where(hit, -1.0, a)
    sup_ref[...] = jnp.concatenate(idxs, axis=0)


def _solve5(G, b):
    k = _KS
    for p in range(k):
        inv = 1.0 / G[p][p]
        for r in range(p + 1, k):
            f = G[r][p] * inv
            for c2 in range(p + 1, k):
                G[r][c2] = G[r][c2] - f * G[p][c2]
            b[r] = b[r] - f * b[p]
    v = [None] * k
    for p in range(k - 1, -1, -1):
        acc = b[p]
        for c2 in range(p + 1, k):
            acc = acc - G[p][c2] * v[c2]
        v[p] = acc / G[p][p]
    return [jnp.where(jnp.abs(val) <= 3.0e38, val, 0.0) for val in v]


def _sc_body(sup_hbm, x_hbm, dt_hbm, xhat_hbm, losspart_hbm,
             sup_v, at0, at1, at2, at3, at4, x_v, stage_v, out_v, loss_v,
             sem):
    C = 32
    HC = C // 2
    wid = lax.axis_index("s") * _NC + lax.axis_index("c")
    n_tok = x_hbm.shape[0] // C
    tok_per_w = n_tok // _NW
    atoms_v = (at0, at1, at2, at3, at4)
    lane = lax.iota(jnp.int32, _LANES)
    lane16 = lane * _LANES

    off = wid * tok_per_w
    for i in range(_KS):
        pltpu.sync_copy(sup_hbm.at[pl.ds(i * n_tok + off, tok_per_w)],
                        sup_v.at[pl.ds(i * tok_per_w, tok_per_w)])
    copies = []
    for i in range(_KS):
        for q in range(tok_per_w // _CH):
            copies.append(pltpu.async_copy(
                dt_hbm.at[sup_v.at[pl.ds(i * tok_per_w + q * _CH, _CH)]],
                atoms_v[i].at[pl.ds(q * _CH, _CH), :], sem))
    copies.append(pltpu.async_copy(
        x_hbm.at[pl.ds(off * C, tok_per_w * C)], x_v, sem))
    for cp in copies:
        cp.wait()

    def group_body(g, acc):
            base = g * _LANES
            for t in range(_LANES):
                tok = base + t
                for i in range(_KS):
                    lo = atoms_v[i][tok, pl.ds(0, _LANES)]
                    hi = atoms_v[i][tok, pl.ds(_LANES, _LANES)]
                    plsc.store_scatter(stage_v, [lane16 + (i * C * _LANES + t)], lo)
                    plsc.store_scatter(
                        stage_v, [lane16 + (i * C * _LANES + HC * _LANES + t)], hi)
            tok32 = (base + lane) * C
            G = [[None] * _KS for _ in range(_KS)]
            rhs = [None] * _KS
            for c in range(C):
                xc = plsc.load_gather(x_v, [tok32 + c])
                ac = [stage_v[pl.ds((i * C + c) * _LANES, _LANES)]
                      for i in range(_KS)]
                for i in range(_KS):
                    rhs[i] = ac[i] * xc if rhs[i] is None else rhs[i] + ac[i] * xc
                    for j in range(i, _KS):
                        prod = ac[i] * ac[j]
                        G[i][j] = prod if G[i][j] is None else G[i][j] + prod
            for i in range(_KS):
                G[i][i] = G[i][i] + _EPS
                for j in range(i + 1, _KS):
                    G[j][i] = G[i][j]
            v = _solve5(G, rhs)
            for c in range(C):
                ac = [stage_v[pl.ds((i * C + c) * _LANES, _LANES)]
                      for i in range(_KS)]
                xh = v[0] * ac[0]
                for i in range(1, _KS):
                    xh = xh + v[i] * ac[i]
                plsc.store_scatter(out_v, [tok32 + c], xh)
                d = xh - plsc.load_gather(x_v, [tok32 + c])
                acc = acc + d * d
            return acc

    loss_acc = lax.fori_loop(0, tok_per_w // _LANES, group_body,
                             jnp.zeros((_LANES,), jnp.float32))
    pltpu.sync_copy(out_v, xhat_hbm.at[pl.ds(off * C, tok_per_w * C)])
    loss_v[...] = loss_acc
    pltpu.sync_copy(loss_v, losspart_hbm.at[wid])


def kernel(z, dictionary):
    B, C, H, W = z.shape
    K = dictionary.shape[1]
    N = B * H * W
    XT = jnp.transpose(z, (1, 0, 2, 3)).reshape(C, N)
    X = XT.T
    DT = dictionary.T
    TB = 1024
    NH = N // 4

    def tc_half(xt_half):
        return pl.pallas_call(
            _tc_select,
            grid=(NH // TB,),
            in_specs=[
                pl.BlockSpec((C, TB), lambda i: (0, i)),
                pl.BlockSpec((C, K), lambda i: (0, 0)),
            ],
            out_specs=pl.BlockSpec((_KS, TB), lambda i: (0, i)),
            out_shape=jax.ShapeDtypeStruct((_KS, NH), jnp.int32),
        )(xt_half, dictionary)

    sc_fn = functools.partial(
        pl.kernel,
        mesh=plsc.VectorSubcoreMesh(core_axis_name="c", subcore_axis_name="s"),
        compiler_params=pltpu.CompilerParams(
            needs_layout_passes=False, use_tc_tiling_on_sc=False),
        out_type=[
            jax.ShapeDtypeStruct((NH * C,), jnp.float32),
            jax.ShapeDtypeStruct((_NW, _LANES), jnp.float32),
        ],
        scratch_types=[
            pltpu.VMEM((_KS * (NH // _NW),), jnp.int32),
        ] + [pltpu.VMEM((NH // _NW, C), jnp.float32) for _ in range(_KS)] + [
            pltpu.VMEM(((NH // _NW) * C,), jnp.float32),
            pltpu.VMEM((_KS * C * _LANES,), jnp.float32),
            pltpu.VMEM(((NH // _NW) * C,), jnp.float32),
            pltpu.VMEM((_LANES,), jnp.float32),
            pltpu.SemaphoreType.DMA,
        ],
    )(_sc_body)

    X_flat = X.reshape(N * C)
    nsl = N // NH
    sups = [tc_half(XT[:, s * NH:(s + 1) * NH]) for s in range(nsl)]
    parts = [sc_fn(sups[s].reshape(_KS * NH),
                   X_flat[s * NH * C:(s + 1) * NH * C], DT)
             for s in range(nsl)]
    xhat = jnp.concatenate([p[0] for p in parts])

    quant = jnp.transpose(xhat.reshape(B, H, W, C), (0, 3, 1, 2))
    loss_sum = parts[0][1].sum()
    for p in parts[1:]:
        loss_sum = loss_sum + p[1].sum()
    loss = loss_sum * (1.0 + _COMMIT) / (N * C)
    return quant, loss


# speedup vs baseline: 1.1651x; 1.1651x over previous
import functools

import jax
import jax.numpy as jnp
from jax import lax
from jax.experimental import pallas as pl
from jax.experimental.pallas import tpu as pltpu
from jax.experimental.pallas import tpu_sc as plsc

_EPS = 1e-10
_KS = 5
_COMMIT = 0.25

_NC = 2
_NS = 16
_NW = _NC * _NS
_LANES = 16
_CH = 128


def _tc_select(xt_ref, d_ref, sup_ref):
    C, TB = xt_ref.shape
    K = d_ref.shape[1]
    xbt = xt_ref[...]
    dmat = d_ref[...]
    corr = lax.dot_general(
        dmat, xbt, (((0,), (0,)), ((), ())),
        preferred_element_type=jnp.float32)
    a = jnp.abs(corr)
    kiota = lax.broadcasted_iota(jnp.int32, (1, K), 1)
    digits = jnp.concatenate([kiota // 32, kiota % 32], axis=0)
    digits_bf = digits.astype(jnp.bfloat16)
    idxs = []
    for _ in range(_KS):
        m = jnp.max(a, axis=0, keepdims=True)
        hit = a >= m
        hitf = hit.astype(jnp.bfloat16)
        dig = lax.dot_general(
            digits_bf, hitf, (((1,), (0,)), ((), ())),
            preferred_element_type=jnp.float32)
        idx = dig[0:1, :].astype(jnp.int32) * 32 + dig[1:2, :].astype(jnp.int32)
        idxs.append(idx)
        a = jnp.where(hit, -1.0, a)
    sup_ref[...] = jnp.concatenate(idxs, axis=0)


def _solve5(G, b):
    k = _KS
    for p in range(k):
        inv = 1.0 / G[p][p]
        for r in range(p + 1, k):
            f = G[r][p] * inv
            for c2 in range(p + 1, k):
                G[r][c2] = G[r][c2] - f * G[p][c2]
            b[r] = b[r] - f * b[p]
    v = [None] * k
    for p in range(k - 1, -1, -1):
        acc = b[p]
        for c2 in range(p + 1, k):
            acc = acc - G[p][c2] * v[c2]
        v[p] = acc / G[p][p]
    return [jnp.where(jnp.abs(val) <= 3.0e38, val, 0.0) for val in v]


def _sc_body(sup_hbm, x_hbm, dt_hbm, xhat_hbm, losspart_hbm,
             sup_v, at0, at1, at2, at3, at4, x_v, stage_v, out_v, loss_v,
             sem):
    C = 32
    HC = C // 2
    wid = lax.axis_index("s") * _NC + lax.axis_index("c")
    n_tok = x_hbm.shape[0] // C
    tok_per_w = n_tok // _NW
    atoms_v = (at0, at1, at2, at3, at4)
    lane = lax.iota(jnp.int32, _LANES)
    lane16 = lane * _LANES

    off = wid * tok_per_w
    hw = 1024
    b_rel = off // hw
    r0 = off % hw
    for i in range(_KS):
        pltpu.sync_copy(sup_hbm.at[pl.ds(i * n_tok + off, tok_per_w)],
                        sup_v.at[pl.ds(i * tok_per_w, tok_per_w)])
    copies = []
    for i in range(_KS):
        for q in range(tok_per_w // _CH):
            copies.append(pltpu.async_copy(
                dt_hbm.at[sup_v.at[pl.ds(i * tok_per_w + q * _CH, _CH)]],
                atoms_v[i].at[pl.ds(q * _CH, _CH), :], sem))
    for c in range(C):
        copies.append(pltpu.async_copy(
            x_hbm.at[pl.ds(b_rel * C * hw + c * hw + r0, tok_per_w)],
            x_v.at[pl.ds(c * tok_per_w, tok_per_w)], sem))
    for cp in copies:
        cp.wait()

    def group_body(g, acc):
            base = g * _LANES
            for t in range(_LANES):
                tok = base + t
                for i in range(_KS):
                    lo = atoms_v[i][tok, pl.ds(0, _LANES)]
                    hi = atoms_v[i][tok, pl.ds(_LANES, _LANES)]
                    plsc.store_scatter(stage_v, [lane16 + (i * C * _LANES + t)], lo)
                    plsc.store_scatter(
                        stage_v, [lane16 + (i * C * _LANES + HC * _LANES + t)], hi)
            G = [[None] * _KS for _ in range(_KS)]
            rhs = [None] * _KS
            for c in range(C):
                xc = x_v[pl.ds(c * tok_per_w + base, _LANES)]
                ac = [stage_v[pl.ds((i * C + c) * _LANES, _LANES)]
                      for i in range(_KS)]
                for i in range(_KS):
                    rhs[i] = ac[i] * xc if rhs[i] is None else rhs[i] + ac[i] * xc
                    for j in range(i, _KS):
                        prod = ac[i] * ac[j]
                        G[i][j] = prod if G[i][j] is None else G[i][j] + prod
            for i in range(_KS):
                G[i][i] = G[i][i] + _EPS
                for j in range(i + 1, _KS):
                    G[j][i] = G[i][j]
            v = _solve5(G, rhs)
            for c in range(C):
                ac = [stage_v[pl.ds((i * C + c) * _LANES, _LANES)]
                      for i in range(_KS)]
                xh = v[0] * ac[0]
                for i in range(1, _KS):
                    xh = xh + v[i] * ac[i]
                d = xh - x_v[pl.ds(c * tok_per_w + base, _LANES)]
                out_v[pl.ds(c * tok_per_w + base, _LANES)] = xh
                acc = acc + d * d
            return acc

    loss_acc = lax.fori_loop(0, tok_per_w // _LANES, group_body,
                             jnp.zeros((_LANES,), jnp.float32))
    for c in range(C):
        pltpu.sync_copy(out_v.at[pl.ds(c * tok_per_w, tok_per_w)],
                        xhat_hbm.at[pl.ds(b_rel * C * hw + c * hw + r0,
                                          tok_per_w)])
    loss_v[...] = loss_acc
    pltpu.sync_copy(loss_v, losspart_hbm.at[wid])


def kernel(z, dictionary):
    B, C, H, W = z.shape
    K = dictionary.shape[1]
    N = B * H * W
    XT = jnp.transpose(z, (1, 0, 2, 3)).reshape(C, N)
    DT = dictionary.T
    TB = 1024
    NH = N // 2

    def tc_half(xt_half):
        return pl.pallas_call(
            _tc_select,
            grid=(NH // TB,),
            in_specs=[
                pl.BlockSpec((C, TB), lambda i: (0, i)),
                pl.BlockSpec((C, K), lambda i: (0, 0)),
            ],
            out_specs=pl.BlockSpec((_KS, TB), lambda i: (0, i)),
            out_shape=jax.ShapeDtypeStruct((_KS, NH), jnp.int32),
        )(xt_half, dictionary)

    sc_fn = functools.partial(
        pl.kernel,
        mesh=plsc.VectorSubcoreMesh(core_axis_name="c", subcore_axis_name="s"),
        compiler_params=pltpu.CompilerParams(
            needs_layout_passes=False, use_tc_tiling_on_sc=False),
        out_type=[
            jax.ShapeDtypeStruct((NH * C,), jnp.float32),
            jax.ShapeDtypeStruct((_NW, _LANES), jnp.float32),
        ],
        scratch_types=[
            pltpu.VMEM((_KS * (NH // _NW),), jnp.int32),
        ] + [pltpu.VMEM((NH // _NW, C), jnp.float32) for _ in range(_KS)] + [
            pltpu.VMEM(((NH // _NW) * C,), jnp.float32),
            pltpu.VMEM((_KS * C * _LANES,), jnp.float32),
            pltpu.VMEM(((NH // _NW) * C,), jnp.float32),
            pltpu.VMEM((_LANES,), jnp.float32),
            pltpu.SemaphoreType.DMA,
        ],
    )(_sc_body)

    z_flat = z.reshape(N * C)
    nsl = N // NH
    sups = [tc_half(XT[:, s * NH:(s + 1) * NH]) for s in range(nsl)]
    parts = [sc_fn(sups[s].reshape(_KS * NH),
                   z_flat[s * NH * C:(s + 1) * NH * C], DT)
             for s in range(nsl)]
    xhat = jnp.concatenate([p[0] for p in parts])

    quant = xhat.reshape(B, C, H, W)
    loss_sum = parts[0][1].sum()
    for p in parts[1:]:
        loss_sum = loss_sum + p[1].sum()
    loss = loss_sum * (1.0 + _COMMIT) / (N * C)
    return quant, loss
